# baseline (speedup 1.0000x reference)
import functools

import jax
import jax.numpy as jnp
from jax import lax
from jax.experimental import pallas as pl
from jax.experimental.pallas import tpu as pltpu
from jax.experimental.pallas import tpu_sc as plsc

EMB = 512
HID = 1024
EPS = 1e-07

_NC = 2
_NS = 16
_NW = _NC * _NS
_CHUNK = 64
_K = 4


def _sc_gather_body(ids_hbm, table_hbm, out_hbm, idx_v, buf0, buf1, s0, s1):
    n_tok = ids_hbm.shape[0]
    tok_per_w = n_tok // _NW
    n_chunks = tok_per_w // _CHUNK
    wid = lax.axis_index("s") * _NC + lax.axis_index("c")
    base = wid * tok_per_w
    pltpu.sync_copy(ids_hbm.at[pl.ds(base, tok_per_w)], idx_v)
    bufs = (buf0, buf1)
    sems = (s0, s1)
    gathers = [None, None]
    gathers[0] = pltpu.async_copy(
        table_hbm.at[idx_v.at[pl.ds(0, _CHUNK)]], bufs[0], sems[0])
    for c in range(n_chunks):
        if c + 1 < n_chunks:
            gathers[(c + 1) % 2] = pltpu.async_copy(
                table_hbm.at[idx_v.at[pl.ds((c + 1) * _CHUNK, _CHUNK)]],
                bufs[(c + 1) % 2], sems[(c + 1) % 2])
        gathers[c % 2].wait()
        pltpu.sync_copy(bufs[c % 2],
                        out_hbm.at[pl.ds(base + c * _CHUNK, _CHUNK)])


def _sc_gather(ids_flat, word_table):
    n_tok = ids_flat.shape[0]
    tok_per_w = n_tok // _NW
    mesh = plsc.VectorSubcoreMesh(core_axis_name="c", subcore_axis_name="s")
    k = functools.partial(
        pl.kernel,
        mesh=mesh,
        out_type=jax.ShapeDtypeStruct((n_tok, EMB), jnp.float32),
        scratch_types=[
            pltpu.VMEM((tok_per_w,), jnp.int32),
            pltpu.VMEM((_CHUNK, EMB), jnp.float32),
            pltpu.VMEM((_CHUNK, EMB), jnp.float32),
            pltpu.SemaphoreType.DMA,
            pltpu.SemaphoreType.DMA,
        ],
    )(_sc_gather_body)
    return k(ids_flat, word_table)


def _tc_body_first(x_ref, pos_ref, w_ref, g_ref, b_ref, o_ref):
    x = (x_ref[...] + pos_ref[...]).astype(jnp.bfloat16)
    h = jnp.dot(x, w_ref[...], preferred_element_type=jnp.float32)
    mu = jnp.mean(h, axis=-1, keepdims=True)
    var = jnp.mean((h - mu) ** 2, axis=-1, keepdims=True)
    o_ref[...] = (h - mu) * lax.rsqrt(var + EPS) * g_ref[...] + b_ref[...]


def _tc_body_chained(prev_ref, x_ref, pos_ref, w_ref, g_ref, b_ref, o_ref):
    del prev_ref
    _tc_body_first(x_ref, pos_ref, w_ref, g_ref, b_ref, o_ref)


def _tc_chunk(k, n_tok, nbatch, sblk_per_chunk, embeds_k, pos_table, wb, g2,
              b2, prev):
    blk = 256
    blocks_per_batch = n_tok // nbatch // blk
    x_spec = pl.BlockSpec((blk, EMB),
                          lambda j, b: (b * sblk_per_chunk + j, 0))
    pos_spec = pl.BlockSpec((blk, EMB),
                            lambda j, b: (k * sblk_per_chunk + j, 0))
    w_spec = pl.BlockSpec((EMB, HID), lambda j, b: (0, 0))
    v_spec = pl.BlockSpec((1, HID), lambda j, b: (0, 0))
    out_spec = pl.BlockSpec(
        (blk, HID),
        lambda j, b: (b * blocks_per_batch + k * sblk_per_chunk + j, 0))
    out_shape = jax.ShapeDtypeStruct((n_tok, HID), jnp.float32)
    grid = (sblk_per_chunk, nbatch)
    if prev is None:
        return pl.pallas_call(
            _tc_body_first,
            grid=grid,
            in_specs=[x_spec, pos_spec, w_spec, v_spec, v_spec],
            out_specs=out_spec,
            out_shape=out_shape,
        )(embeds_k, pos_table, wb, g2, b2)
    return pl.pallas_call(
        _tc_body_chained,
        grid=grid,
        in_specs=[pl.BlockSpec(memory_space=pl.MemorySpace.ANY),
                  x_spec, pos_spec, w_spec, v_spec, v_spec],
        out_specs=out_spec,
        out_shape=out_shape,
        input_output_aliases={0: 0},
    )(prev, embeds_k, pos_table, wb, g2, b2)


def kernel(input_ids, word_table, pos_table, proj_w, ln_gamma, ln_beta):
    bsz, seq_len = input_ids.shape
    n_tok = bsz * seq_len
    ids = input_ids.astype(jnp.int32)
    sch = seq_len // _K
    sblk_per_chunk = sch // 256
    wb = proj_w.astype(jnp.bfloat16)
    g2 = ln_gamma.reshape(1, HID)
    b2 = ln_beta.reshape(1, HID)
    embeds = [
        _sc_gather(ids[:, k * sch:(k + 1) * sch].reshape(-1), word_table)
        for k in range(_K)
    ]
    o = None
    for k in range(_K):
        o = _tc_chunk(k, n_tok, bsz, sblk_per_chunk, embeds[k], pos_table,
                      wb, g2, b2, o)
    return o.reshape(bsz, seq_len, HID)


# speedup vs baseline: 1.1724x; 1.1724x over previous
import functools

import jax
import jax.numpy as jnp
from jax import lax
from jax.experimental import pallas as pl
from jax.experimental.pallas import tpu as pltpu
from jax.experimental.pallas import tpu_sc as plsc

EMB = 512
HID = 1024
EPS = 1e-07

_NC = 2
_NS = 16
_NW = _NC * _NS
_K = 4


def _sc_gather_body(ids_hbm, table_hbm, out_hbm, idx_v, buf_v, sem, *, k,
                    sch, seq_len, bsz):
    tok_per_w = (sch * bsz) // _NW
    wpb = _NW // bsz
    wid = lax.axis_index("s") * _NC + lax.axis_index("c")
    myb = wid // wpb
    myj = wid % wpb
    base_in = myb * seq_len + k * sch + myj * tok_per_w
    pltpu.sync_copy(ids_hbm.at[pl.ds(base_in, tok_per_w)], idx_v)
    pltpu.async_copy(table_hbm.at[idx_v], buf_v, sem).wait()
    pltpu.sync_copy(buf_v, out_hbm.at[pl.ds(wid * tok_per_w, tok_per_w)])


def _sc_gather_chunk(ids_flat, word_table, k, sch, seq_len, bsz):
    tok_per_w = (sch * bsz) // _NW
    mesh = plsc.VectorSubcoreMesh(core_axis_name="c", subcore_axis_name="s")
    body = functools.partial(_sc_gather_body, k=k, sch=sch, seq_len=seq_len,
                             bsz=bsz)
    kern = functools.partial(
        pl.kernel,
        mesh=mesh,
        out_type=jax.ShapeDtypeStruct((sch * bsz, EMB), jnp.float32),
        scratch_types=[
            pltpu.VMEM((tok_per_w,), jnp.int32),
            pltpu.VMEM((tok_per_w, EMB), jnp.float32),
            pltpu.SemaphoreType.DMA,
        ],
    )(body)
    return kern(ids_flat, word_table)


def _tc_body_first(x_ref, pos_ref, w_ref, g_ref, b_ref, o_ref):
    x = (x_ref[...] + pos_ref[...]).astype(jnp.bfloat16)
    h = jnp.dot(x, w_ref[...], preferred_element_type=jnp.float32)
    mu = jnp.mean(h, axis=-1, keepdims=True)
    var = jnp.mean((h - mu) ** 2, axis=-1, keepdims=True)
    o_ref[...] = (h - mu) * lax.rsqrt(var + EPS) * g_ref[...] + b_ref[...]


def _tc_body_chained(prev_ref, x_ref, pos_ref, w_ref, g_ref, b_ref, o_ref):
    del prev_ref
    _tc_body_first(x_ref, pos_ref, w_ref, g_ref, b_ref, o_ref)


def _tc_chunk(k, n_tok, nbatch, sch, embeds_k, pos_table, wb, g2, b2, prev):
    blk = sch
    blocks_per_batch = (n_tok // nbatch) // blk
    x_spec = pl.BlockSpec((blk, EMB), lambda j, b: (b, 0))
    pos_spec = pl.BlockSpec((blk, EMB), lambda j, b: (k, 0))
    w_spec = pl.BlockSpec((EMB, HID), lambda j, b: (0, 0))
    v_spec = pl.BlockSpec((1, HID), lambda j, b: (0, 0))
    out_spec = pl.BlockSpec(
        (blk, HID), lambda j, b: (b * blocks_per_batch + k, 0))
    out_shape = jax.ShapeDtypeStruct((n_tok, HID), jnp.float32)
    grid = (1, nbatch)
    if prev is None:
        return pl.pallas_call(
            _tc_body_first,
            grid=grid,
            in_specs=[x_spec, pos_spec, w_spec, v_spec, v_spec],
            out_specs=out_spec,
            out_shape=out_shape,
        )(embeds_k, pos_table, wb, g2, b2)
    return pl.pallas_call(
        _tc_body_chained,
        grid=grid,
        in_specs=[pl.BlockSpec(memory_space=pl.MemorySpace.ANY),
                  x_spec, pos_spec, w_spec, v_spec, v_spec],
        out_specs=out_spec,
        out_shape=out_shape,
        input_output_aliases={0: 0},
    )(prev, embeds_k, pos_table, wb, g2, b2)


def kernel(input_ids, word_table, pos_table, proj_w, ln_gamma, ln_beta):
    bsz, seq_len = input_ids.shape
    n_tok = bsz * seq_len
    ids_flat = input_ids.reshape(-1).astype(jnp.int32)
    sch = seq_len // _K
    wb = proj_w.astype(jnp.bfloat16)
    g2 = ln_gamma.reshape(1, HID)
    b2 = ln_beta.reshape(1, HID)
    embeds = [
        _sc_gather_chunk(ids_flat, word_table, k, sch, seq_len, bsz)
        for k in range(_K)
    ]
    o = None
    for k in range(_K):
        o = _tc_chunk(k, n_tok, bsz, sch, embeds[k], pos_table, wb, g2, b2, o)
    return o.reshape(bsz, seq_len, HID)


# speedup vs baseline: 1.1741x; 1.0014x over previous
import functools

import jax
import jax.numpy as jnp
from jax import lax
from jax.experimental import pallas as pl
from jax.experimental.pallas import tpu as pltpu
from jax.experimental.pallas import tpu_sc as plsc

EMB = 512
HID = 1024
EPS = 1e-07

_NC = 2
_NS = 16
_NW = _NC * _NS
_K = 4


def _sc_gather_body(ids_hbm, table_hbm, out_hbm, idx_v, buf_v, sem, *, k,
                    sch, seq_len, bsz):
    tok_per_w = (sch * bsz) // _NW
    wpb = _NW // bsz
    wid = lax.axis_index("s") * _NC + lax.axis_index("c")
    myb = wid // wpb
    myj = wid % wpb
    col = k * sch + myj * tok_per_w
    pltpu.sync_copy(ids_hbm.at[myb, pl.ds(col, tok_per_w)], idx_v)
    pltpu.async_copy(table_hbm.at[idx_v], buf_v, sem).wait()
    pltpu.sync_copy(buf_v, out_hbm.at[pl.ds(wid * tok_per_w, tok_per_w)])


def _sc_gather_chunk(ids2d, word_table, k, sch, seq_len, bsz):
    tok_per_w = (sch * bsz) // _NW
    mesh = plsc.VectorSubcoreMesh(core_axis_name="c", subcore_axis_name="s")
    body = functools.partial(_sc_gather_body, k=k, sch=sch, seq_len=seq_len,
                             bsz=bsz)
    kern = functools.partial(
        pl.kernel,
        mesh=mesh,
        out_type=jax.ShapeDtypeStruct((sch * bsz, EMB), jnp.float32),
        scratch_types=[
            pltpu.VMEM((tok_per_w,), jnp.int32),
            pltpu.VMEM((tok_per_w, EMB), jnp.float32),
            pltpu.SemaphoreType.DMA,
        ],
    )(body)
    return kern(ids2d, word_table)


def _tc_body_first(x_ref, pos_ref, w_ref, g_ref, b_ref, o_ref):
    x = (x_ref[...] + pos_ref[...]).astype(jnp.bfloat16)
    h = jnp.dot(x, w_ref[...], preferred_element_type=jnp.float32)
    mu = jnp.mean(h, axis=-1, keepdims=True)
    var = jnp.mean((h - mu) ** 2, axis=-1, keepdims=True)
    o_ref[...] = (h - mu) * lax.rsqrt(var + EPS) * g_ref[...] + b_ref[...]


def _tc_body_chained(prev_ref, x_ref, pos_ref, w_ref, g_ref, b_ref, o_ref):
    del prev_ref
    _tc_body_first(x_ref, pos_ref, w_ref, g_ref, b_ref, o_ref)


def _tc_chunk(k, n_tok, nbatch, sch, embeds_k, pos_table, wb, g2, b2, prev):
    blk = sch
    blocks_per_batch = (n_tok // nbatch) // blk
    x_spec = pl.BlockSpec((blk, EMB), lambda j, b: (b, 0))
    pos_spec = pl.BlockSpec((blk, EMB), lambda j, b: (k, 0))
    w_spec = pl.BlockSpec((EMB, HID), lambda j, b: (0, 0))
    v_spec = pl.BlockSpec((1, HID), lambda j, b: (0, 0))
    out_spec = pl.BlockSpec(
        (blk, HID), lambda j, b: (b * blocks_per_batch + k, 0))
    out_shape = jax.ShapeDtypeStruct((n_tok, HID), jnp.float32)
    grid = (1, nbatch)
    if prev is None:
        return pl.pallas_call(
            _tc_body_first,
            grid=grid,
            in_specs=[x_spec, pos_spec, w_spec, v_spec, v_spec],
            out_specs=out_spec,
            out_shape=out_shape,
        )(embeds_k, pos_table, wb, g2, b2)
    return pl.pallas_call(
        _tc_body_chained,
        grid=grid,
        in_specs=[pl.BlockSpec(memory_space=pl.MemorySpace.ANY),
                  x_spec, pos_spec, w_spec, v_spec, v_spec],
        out_specs=out_spec,
        out_shape=out_shape,
        input_output_aliases={0: 0},
    )(prev, embeds_k, pos_table, wb, g2, b2)


def kernel(input_ids, word_table, pos_table, proj_w, ln_gamma, ln_beta):
    bsz, seq_len = input_ids.shape
    n_tok = bsz * seq_len
    ids2d = input_ids.astype(jnp.int32)
    sch = seq_len // _K
    wb = proj_w.astype(jnp.bfloat16)
    g2 = ln_gamma.reshape(1, HID)
    b2 = ln_beta.reshape(1, HID)
    embeds = [
        _sc_gather_chunk(ids2d, word_table, k, sch, seq_len, bsz)
        for k in range(_K)
    ]
    o = None
    for k in range(_K):
        o = _tc_chunk(k, n_tok, bsz, sch, embeds[k], pos_table, wb, g2, b2, o)
    return o.reshape(bsz, seq_len, HID)
